# Initial kernel scaffold; baseline (speedup 1.0000x reference)
import jax
import jax.numpy as jnp
from jax.experimental import pallas as pl


def kernel(features, edge_index, W_self1, W_neigh1, b1, W_self2, W_neigh2, b2, W_fc, b_fc):
    raise NotImplementedError("write your pallas kernel here")


# speedup vs baseline: 3.9285x; 3.9285x over previous
import jax
import jax.numpy as jnp
from jax import lax
from jax.experimental import pallas as pl
from jax.experimental.pallas import tpu as pltpu
from jax.experimental.pallas import tpu_sc as plsc

N_NODES = 10000
N_EDGES = 320000
D_FEAT = 128
N_PART = 8

N_CORES = 2
N_SUBCORES = 16
N_TILES = N_CORES * N_SUBCORES
LANES = 16

CHUNK = 128
EDGES_PER_TILE = -(-N_EDGES // (N_TILES * CHUNK)) * CHUNK
N_CHUNKS = EDGES_PER_TILE // CHUNK
E_PAD = EDGES_PER_TILE * N_TILES

N_PAD = 10240
ROWS_PER_TILE = N_PAD // N_SUBCORES


def _sc_deg_body(dst_hbm, z128_hbm, ones_hbm, deg_out,
                 didx, buf, deg_sh, sem):
    c = lax.axis_index("c")
    s = lax.axis_index("s")
    wid = c * N_SUBCORES + s

    r0 = s * ROWS_PER_TILE
    n_sub = ROWS_PER_TILE // CHUNK

    pltpu.sync_copy(z128_hbm.at[pl.ds(0, CHUNK)], buf)
    for k in range(n_sub):
        pltpu.sync_copy(buf, deg_sh.at[pl.ds(r0 + k * CHUNK, CHUNK)])
    pltpu.sync_copy(ones_hbm.at[pl.ds(0, CHUNK)], buf)
    plsc.subcore_barrier()

    base = wid * EDGES_PER_TILE

    def step(j, carry):
        off = base + j * CHUNK
        pltpu.sync_copy(dst_hbm.at[pl.ds(off, CHUNK)], didx)
        pltpu.sync_copy(buf, deg_sh.at[didx], add=True)
        return carry

    lax.fori_loop(0, N_CHUNKS, step, 0)
    plsc.subcore_barrier()

    for k in range(n_sub):
        pltpu.sync_copy(deg_sh.at[pl.ds(r0 + k * CHUNK, CHUNK)], buf)
        pltpu.sync_copy(buf, deg_out.at[pl.ds(c * N_PAD + r0 + k * CHUNK, CHUNK)])


def _sc_body_nodeg(x_hbm, src_hbm, dst_hbm, z128_hbm, agg_out,
                   sidx, didx, rows, acc_sh, sem):
    c = lax.axis_index("c")
    s = lax.axis_index("s")
    wid = c * N_SUBCORES + s

    r0 = s * ROWS_PER_TILE
    n_sub = ROWS_PER_TILE // CHUNK
    pltpu.sync_copy(z128_hbm.at[pl.ds(0, CHUNK)], rows)
    for k in range(n_sub):
        pltpu.sync_copy(rows, acc_sh.at[pl.ds(r0 + k * CHUNK, CHUNK)])
    plsc.subcore_barrier()

    base = wid * EDGES_PER_TILE

    def step(j, carry):
        off = base + j * CHUNK
        pltpu.sync_copy(src_hbm.at[pl.ds(off, CHUNK)], sidx)
        pltpu.sync_copy(dst_hbm.at[pl.ds(off, CHUNK)], didx)
        pltpu.async_copy(x_hbm.at[sidx], rows, sem).wait()
        pltpu.sync_copy(rows, acc_sh.at[didx], add=True)
        return carry

    lax.fori_loop(0, N_CHUNKS, step, 0)
    plsc.subcore_barrier()

    for k in range(n_sub):
        pltpu.sync_copy(acc_sh.at[pl.ds(r0 + k * CHUNK, CHUNK)], rows)
        pltpu.sync_copy(rows, agg_out.at[pl.ds(c * N_PAD + r0 + k * CHUNK, CHUNK)])


def _sc_mesh():
    return plsc.VectorSubcoreMesh(core_axis_name="c", subcore_axis_name="s",
                                  num_cores=N_CORES, num_subcores=N_SUBCORES)


_sc_deg = pl.kernel(
    _sc_deg_body,
    out_type=[jax.ShapeDtypeStruct((N_CORES * N_PAD, 128), jnp.float32)],
    mesh=_sc_mesh(),
    scratch_types=[
        pltpu.VMEM((CHUNK,), jnp.int32),
        pltpu.VMEM((CHUNK, 128), jnp.float32),
        pltpu.VMEM_SHARED((N_PAD, 128), jnp.float32),
        pltpu.SemaphoreType.DMA,
    ],
)

_sc_agg = pl.kernel(
    _sc_body_nodeg,
    out_type=[jax.ShapeDtypeStruct((N_CORES * N_PAD, 128), jnp.float32)],
    mesh=_sc_mesh(),
    scratch_types=[
        pltpu.VMEM((CHUNK,), jnp.int32),
        pltpu.VMEM((CHUNK,), jnp.int32),
        pltpu.VMEM((CHUNK, 128), jnp.float32),
        pltpu.VMEM_SHARED((N_PAD, 128), jnp.float32),
        pltpu.SemaphoreType.DMA,
    ],
)


ROW_BLOCK = 1000
N_ROW_BLOCKS = N_NODES // ROW_BLOCK


def _sage_block(x_ref, p0_ref, p1_ref, d_ref, ws_ref, wn_ref, b_ref):
    rdeg = 1.0 / jnp.maximum(d_ref[...], 1.0)
    mean = (p0_ref[...] + p1_ref[...]) * rdeg
    h = (jnp.dot(x_ref[...], ws_ref[...], preferred_element_type=jnp.float32)
         + jnp.dot(mean, wn_ref[...], preferred_element_type=jnp.float32)
         + b_ref[...])
    return jnp.maximum(h, 0.0)


def _sage_layer_kernel(x_ref, p0_ref, p1_ref, d_ref, ws_ref, wn_ref, b_ref,
                       o_ref):
    o_ref[...] = _sage_block(x_ref, p0_ref, p1_ref, d_ref, ws_ref, wn_ref, b_ref)


def _sage_head_kernel(x_ref, p0_ref, p1_ref, d_ref, ws_ref, wn_ref, b_ref,
                      wf_ref, bf_ref, o_ref):
    h = _sage_block(x_ref, p0_ref, p1_ref, d_ref, ws_ref, wn_ref, b_ref)
    logits = (jnp.dot(h, wf_ref[...], preferred_element_type=jnp.float32)
              + bf_ref[...])
    m = jnp.max(logits, axis=1, keepdims=True)
    e = jnp.exp(logits - m)
    o_ref[...] = e / jnp.sum(e, axis=1, keepdims=True)


def _row_spec(width):
    return pl.BlockSpec((ROW_BLOCK, width), lambda i: (i, 0))


def _full_spec(shape):
    return pl.BlockSpec(shape, lambda i: tuple(0 for _ in shape))


_sage_layer = pl.pallas_call(
    _sage_layer_kernel,
    grid=(N_ROW_BLOCKS,),
    in_specs=[
        _row_spec(128), _row_spec(128), _row_spec(128), _row_spec(1),
        _full_spec((128, 128)), _full_spec((128, 128)), _full_spec((1, 128)),
    ],
    out_specs=_row_spec(128),
    out_shape=jax.ShapeDtypeStruct((N_NODES, 128), jnp.float32),
)

_sage_head = pl.pallas_call(
    _sage_head_kernel,
    grid=(N_ROW_BLOCKS,),
    in_specs=[
        _row_spec(128), _row_spec(128), _row_spec(128), _row_spec(1),
        _full_spec((128, 128)), _full_spec((128, 128)), _full_spec((1, 128)),
        _full_spec((128, N_PART)), _full_spec((1, N_PART)),
    ],
    out_specs=_row_spec(N_PART),
    out_shape=jax.ShapeDtypeStruct((N_NODES, N_PART), jnp.float32),
)


def kernel(features, edge_index, W_self1, W_neigh1, b1, W_self2, W_neigh2, b2,
           W_fc, b_fc):
    src = edge_index[0].astype(jnp.int32)
    dst = edge_index[1].astype(jnp.int32)
    pad = E_PAD - N_EDGES
    src_p = jnp.concatenate([src, jnp.zeros((pad,), jnp.int32)])
    dst_p = jnp.concatenate([dst, jnp.full((pad,), N_NODES, jnp.int32)])

    z128 = jnp.zeros((N_PAD, 128), jnp.float32)
    ones128 = jnp.ones((CHUNK, 128), jnp.float32)

    (deg128,) = _sc_deg(dst_p, z128, ones128)
    deg = (deg128[0:N_NODES, 0:1] + deg128[N_PAD:N_PAD + N_NODES, 0:1])

    (agg1,) = _sc_agg(features, src_p, dst_p, z128)
    p0 = agg1[0:N_NODES]
    p1 = agg1[N_PAD:N_PAD + N_NODES]

    h1 = _sage_layer(features, p0, p1, deg, W_self1, W_neigh1,
                     b1.reshape(1, 128))

    (agg2,) = _sc_agg(h1, src_p, dst_p, z128)
    q0 = agg2[0:N_NODES]
    q1 = agg2[N_PAD:N_PAD + N_NODES]

    return _sage_head(h1, q0, q1, deg, W_self2, W_neigh2,
                      b2.reshape(1, 128), W_fc, b_fc.reshape(1, N_PART))
